# baseline (speedup 1.0000x reference)
import functools

import jax
import jax.numpy as jnp
from jax import lax
from jax.experimental import pallas as pl
from jax.experimental.pallas import tpu as pltpu
from jax.experimental.pallas import tpu_sc as plsc

N_ROWS = 65536
N_COLS = 1024
ROW_BLOCK = 512


def _argmax_body(k_ref, idx_ref):
    x = k_ref[...]
    m = jnp.max(x, axis=1, keepdims=True)
    colf = lax.broadcasted_iota(jnp.int32, x.shape, 1).astype(jnp.float32)
    amax = jnp.min(jnp.where(x == m, colf, 2048.0), axis=1).astype(jnp.int32)
    row = pl.program_id(0) * ROW_BLOCK + lax.iota(jnp.int32, ROW_BLOCK)
    idx_ref[...] = (
        (row // 8) * 8192
        + (amax // 128) * 1024
        + (row % 8) * 128
        + (amax % 128)
    )


def _tc_argmax(k):
    return pl.pallas_call(
        _argmax_body,
        grid=(N_ROWS // ROW_BLOCK,),
        in_specs=[pl.BlockSpec((ROW_BLOCK, N_COLS), lambda i: (i, 0))],
        out_specs=pl.BlockSpec((ROW_BLOCK,), lambda i: (i,)),
        out_shape=jax.ShapeDtypeStruct((N_ROWS,), jnp.int32),
    )(k)


def _make_sc_gather():
    info = plsc.get_sparse_core_info()
    nw = info.num_cores * info.num_subcores
    b_per_w = N_ROWS // nw
    mesh = plsc.VectorSubcoreMesh(core_axis_name="c", subcore_axis_name="s")

    @functools.partial(
        pl.kernel,
        mesh=mesh,
        out_type=jax.ShapeDtypeStruct((N_ROWS,), jnp.float32),
        scratch_types=[
            pltpu.VMEM((b_per_w,), jnp.int32),
            pltpu.VMEM((b_per_w,), jnp.float32),
            pltpu.SemaphoreType.DMA,
        ],
    )
    def gather(vflat_hbm, idx_hbm, out_hbm, idx_v, vals_v, sem):
        wid = lax.axis_index("s") * info.num_cores + lax.axis_index("c")
        base = wid * b_per_w
        pltpu.sync_copy(idx_hbm.at[pl.ds(base, b_per_w)], idx_v)
        pltpu.async_copy(vflat_hbm.at[idx_v], vals_v, sem).wait()
        pltpu.sync_copy(vals_v, out_hbm.at[pl.ds(base, b_per_w)])

    return gather


_sc_gather = _make_sc_gather()


def kernel(v, k):
    flat_idx = _tc_argmax(k)
    v_view = (
        v.reshape(N_ROWS // 8, 8, N_COLS // 128, 128)
        .transpose(0, 2, 1, 3)
        .reshape(-1)
    )
    return _sc_gather(v_view, flat_idx)


# speedup vs baseline: 1.4560x; 1.4560x over previous
import functools

import jax
import jax.numpy as jnp
from jax import lax
from jax.experimental import pallas as pl
from jax.experimental.pallas import tpu as pltpu
from jax.experimental.pallas import tpu_sc as plsc

N_ROWS = 65536
N_COLS = 1024
ROW_BLOCK = 2048


def _argmax_body(k_ref, idx_ref):
    x = k_ref[...]
    m = jnp.max(x, axis=1, keepdims=True)
    colf = lax.broadcasted_iota(jnp.int32, x.shape, 1).astype(jnp.float32)
    amax = jnp.min(jnp.where(x == m, colf, 2048.0), axis=1).astype(jnp.int32)
    row = pl.program_id(0) * ROW_BLOCK + lax.iota(jnp.int32, ROW_BLOCK)
    idx_ref[...] = (
        (row // 8) * 8192
        + (amax // 128) * 1024
        + (row % 8) * 128
        + (amax % 128)
    )


def _tc_argmax(k):
    return pl.pallas_call(
        _argmax_body,
        grid=(N_ROWS // ROW_BLOCK,),
        in_specs=[pl.BlockSpec((ROW_BLOCK, N_COLS), lambda i: (i, 0))],
        out_specs=pl.BlockSpec((ROW_BLOCK,), lambda i: (i,)),
        out_shape=jax.ShapeDtypeStruct((N_ROWS,), jnp.int32),
    )(k)


def _make_sc_gather():
    info = plsc.get_sparse_core_info()
    nw = info.num_cores * info.num_subcores
    b_per_w = N_ROWS // nw
    mesh = plsc.VectorSubcoreMesh(core_axis_name="c", subcore_axis_name="s")

    @functools.partial(
        pl.kernel,
        mesh=mesh,
        out_type=jax.ShapeDtypeStruct((N_ROWS,), jnp.float32),
        scratch_types=[
            pltpu.VMEM((b_per_w,), jnp.int32),
            pltpu.VMEM((b_per_w,), jnp.float32),
            pltpu.SemaphoreType.DMA,
        ],
    )
    def gather(vflat_hbm, idx_hbm, out_hbm, idx_v, vals_v, sem):
        wid = lax.axis_index("s") * info.num_cores + lax.axis_index("c")
        base = wid * b_per_w
        pltpu.sync_copy(idx_hbm.at[pl.ds(base, b_per_w)], idx_v)
        pltpu.async_copy(vflat_hbm.at[idx_v], vals_v, sem).wait()
        pltpu.sync_copy(vals_v, out_hbm.at[pl.ds(base, b_per_w)])

    return gather


_sc_gather = _make_sc_gather()


def kernel(v, k):
    flat_idx = _tc_argmax(k)
    v_view = (
        v.reshape(N_ROWS // 8, 8, N_COLS // 128, 128)
        .transpose(0, 2, 1, 3)
        .reshape(-1)
    )
    return _sc_gather(v_view, flat_idx)


# speedup vs baseline: 1.5513x; 1.0655x over previous
import functools

import jax
import jax.numpy as jnp
from jax import lax
from jax.experimental import pallas as pl
from jax.experimental.pallas import tpu as pltpu
from jax.experimental.pallas import tpu_sc as plsc

N_ROWS = 65536
N_COLS = 1024
ROW_BLOCK = 4096


def _argmax_body(k_ref, idx_ref):
    x = k_ref[...]
    m = jnp.max(x, axis=1, keepdims=True)
    colf = lax.broadcasted_iota(jnp.int32, x.shape, 1).astype(jnp.float32)
    amax = jnp.min(jnp.where(x == m, colf, 2048.0), axis=1).astype(jnp.int32)
    row = pl.program_id(0) * ROW_BLOCK + lax.iota(jnp.int32, ROW_BLOCK)
    idx_ref[...] = (
        (row // 8) * 8192
        + (amax // 128) * 1024
        + (row % 8) * 128
        + (amax % 128)
    )


def _tc_argmax(k):
    return pl.pallas_call(
        _argmax_body,
        grid=(N_ROWS // ROW_BLOCK,),
        in_specs=[pl.BlockSpec((ROW_BLOCK, N_COLS), lambda i: (i, 0))],
        out_specs=pl.BlockSpec((ROW_BLOCK,), lambda i: (i,)),
        out_shape=jax.ShapeDtypeStruct((N_ROWS,), jnp.int32),
    )(k)


def _make_sc_gather():
    info = plsc.get_sparse_core_info()
    nw = info.num_cores * info.num_subcores
    b_per_w = N_ROWS // nw
    mesh = plsc.VectorSubcoreMesh(core_axis_name="c", subcore_axis_name="s")

    @functools.partial(
        pl.kernel,
        mesh=mesh,
        out_type=jax.ShapeDtypeStruct((N_ROWS,), jnp.float32),
        scratch_types=[
            pltpu.VMEM((b_per_w,), jnp.int32),
            pltpu.VMEM((b_per_w,), jnp.float32),
            pltpu.SemaphoreType.DMA,
        ],
    )
    def gather(vflat_hbm, idx_hbm, out_hbm, idx_v, vals_v, sem):
        wid = lax.axis_index("s") * info.num_cores + lax.axis_index("c")
        base = wid * b_per_w
        pltpu.sync_copy(idx_hbm.at[pl.ds(base, b_per_w)], idx_v)
        pltpu.async_copy(vflat_hbm.at[idx_v], vals_v, sem).wait()
        pltpu.sync_copy(vals_v, out_hbm.at[pl.ds(base, b_per_w)])

    return gather


_sc_gather = _make_sc_gather()


def kernel(v, k):
    flat_idx = _tc_argmax(k)
    v_view = (
        v.reshape(N_ROWS // 8, 8, N_COLS // 128, 128)
        .transpose(0, 2, 1, 3)
        .reshape(-1)
    )
    return _sc_gather(v_view, flat_idx)


# speedup vs baseline: 1.5698x; 1.0119x over previous
import functools

import jax
import jax.numpy as jnp
from jax import lax
from jax.experimental import pallas as pl
from jax.experimental.pallas import tpu as pltpu
from jax.experimental.pallas import tpu_sc as plsc

N_ROWS = 65536
N_COLS = 1024
ROW_BLOCK = 4096


def _argmax_body(k_ref, idx_ref):
    x = k_ref[...]
    m = jnp.max(x, axis=1, keepdims=True)
    colf = lax.broadcasted_iota(jnp.int32, (1, N_COLS), 1).astype(jnp.float32)
    amax = jnp.min(jnp.where(x == m, colf, 2048.0), axis=1)
    idx_ref[...] = amax.astype(jnp.int32)


def _tc_argmax(k):
    return pl.pallas_call(
        _argmax_body,
        grid=(N_ROWS // ROW_BLOCK,),
        in_specs=[pl.BlockSpec((ROW_BLOCK, N_COLS), lambda i: (i, 0))],
        out_specs=pl.BlockSpec((ROW_BLOCK,), lambda i: (i,)),
        out_shape=jax.ShapeDtypeStruct((N_ROWS,), jnp.int32),
        compiler_params=pltpu.CompilerParams(
            vmem_limit_bytes=100 * 1024 * 1024,
        ),
    )(k)


def _make_sc_gather():
    info = plsc.get_sparse_core_info()
    nw = info.num_cores * info.num_subcores
    b_per_w = N_ROWS // nw
    lanes = 16
    mesh = plsc.VectorSubcoreMesh(core_axis_name="c", subcore_axis_name="s")

    @functools.partial(
        pl.kernel,
        mesh=mesh,
        out_type=jax.ShapeDtypeStruct((N_ROWS,), jnp.float32),
        scratch_types=[
            pltpu.VMEM((b_per_w,), jnp.int32),
            pltpu.VMEM((b_per_w,), jnp.float32),
            pltpu.SemaphoreType.DMA,
        ],
    )
    def gather(vflat_hbm, col_hbm, out_hbm, idx_v, vals_v, sem):
        wid = lax.axis_index("s") * info.num_cores + lax.axis_index("c")
        base = wid * b_per_w
        pltpu.sync_copy(col_hbm.at[pl.ds(base, b_per_w)], idx_v)

        def step(i, _):
            c = idx_v[pl.ds(i * lanes, lanes)]
            r = base + i * lanes + lax.iota(jnp.int32, lanes)
            phys = (
                ((r >> 3) << 13)
                + ((c >> 7) << 10)
                + ((r & 7) << 7)
                + (c & 127)
            )
            idx_v[pl.ds(i * lanes, lanes)] = phys
            return 0

        lax.fori_loop(0, b_per_w // lanes, step, 0)

        pltpu.async_copy(vflat_hbm.at[idx_v], vals_v, sem).wait()
        pltpu.sync_copy(vals_v, out_hbm.at[pl.ds(base, b_per_w)])

    return gather


_sc_gather = _make_sc_gather()


def kernel(v, k):
    col_idx = _tc_argmax(k)
    v_view = (
        v.reshape(N_ROWS // 8, 8, N_COLS // 128, 128)
        .transpose(0, 2, 1, 3)
        .reshape(-1)
    )
    return _sc_gather(v_view, col_idx)


# speedup vs baseline: 1.5898x; 1.0127x over previous
import functools

import jax
import jax.numpy as jnp
from jax import lax
from jax.experimental import pallas as pl
from jax.experimental.pallas import tpu as pltpu
from jax.experimental.pallas import tpu_sc as plsc

N_ROWS = 65536
N_COLS = 1024
ROW_BLOCK = 4096
STRIP = 128
LANES = 128


def _argmax_body(k_ref, idx_ref):
    lanef = lax.broadcasted_iota(jnp.int32, (STRIP, LANES), 1).astype(
        jnp.float32
    )
    rows = []
    for s in range(ROW_BLOCK // STRIP):
        r0 = s * STRIP
        mr = k_ref[pl.ds(r0, STRIP), pl.ds(0, LANES)]
        jrf = jnp.zeros((STRIP, LANES), jnp.float32)
        for j in range(1, N_COLS // LANES):
            xj = k_ref[pl.ds(r0, STRIP), pl.ds(j * LANES, LANES)]
            jrf = jnp.where(xj > mr, jnp.float32(j), jrf)
            mr = jnp.maximum(mr, xj)
        m = jnp.max(mr, axis=1, keepdims=True)
        comb = jnp.where(mr == m, jrf * 128.0 + lanef, 2048.0)
        amax = jnp.min(comb, axis=1, keepdims=True)
        rows.append(jnp.transpose(amax))
    idx_ref[...] = jnp.concatenate(rows, axis=0).astype(jnp.int32)


def _tc_argmax(k):
    n_strips = N_ROWS // STRIP
    blk = ROW_BLOCK // STRIP
    return pl.pallas_call(
        _argmax_body,
        grid=(N_ROWS // ROW_BLOCK,),
        in_specs=[pl.BlockSpec((ROW_BLOCK, N_COLS), lambda i: (i, 0))],
        out_specs=pl.BlockSpec((blk, STRIP), lambda i: (i, 0)),
        out_shape=jax.ShapeDtypeStruct((n_strips, STRIP), jnp.int32),
        compiler_params=pltpu.CompilerParams(
            vmem_limit_bytes=100 * 1024 * 1024,
        ),
    )(k)


def _make_sc_gather():
    info = plsc.get_sparse_core_info()
    nw = info.num_cores * info.num_subcores
    b_per_w = N_ROWS // nw
    lanes = 16
    mesh = plsc.VectorSubcoreMesh(core_axis_name="c", subcore_axis_name="s")

    @functools.partial(
        pl.kernel,
        mesh=mesh,
        out_type=jax.ShapeDtypeStruct((N_ROWS,), jnp.float32),
        scratch_types=[
            pltpu.VMEM((b_per_w,), jnp.int32),
            pltpu.VMEM((b_per_w,), jnp.int32),
            pltpu.VMEM((b_per_w,), jnp.float32),
            pltpu.SemaphoreType.DMA,
        ],
    )
    def gather(vflat_hbm, colq_hbm, out_hbm, colq_v, phys_v, vals_v, sem):
        wid = lax.axis_index("s") * info.num_cores + lax.axis_index("c")
        base = wid * b_per_w
        pltpu.sync_copy(colq_hbm.at[pl.ds(base, b_per_w)], colq_v)

        def step(i, _):
            c = colq_v[pl.ds(i * lanes, lanes)]
            r0 = (((i >> 6) << 3) + ((i >> 3) & 7)) * 128 + ((i & 7) << 4)
            r = base + r0 + lax.iota(jnp.int32, lanes)
            phys_v[pl.ds(r0, lanes)] = (
                ((r >> 3) << 13)
                + ((c >> 7) << 10)
                + ((r & 7) << 7)
                + (c & 127)
            )
            return 0

        lax.fori_loop(0, b_per_w // lanes, step, 0)

        pltpu.async_copy(vflat_hbm.at[phys_v], vals_v, sem).wait()
        pltpu.sync_copy(vals_v, out_hbm.at[pl.ds(base, b_per_w)])

    return gather


_sc_gather = _make_sc_gather()


def kernel(v, k):
    col2d = _tc_argmax(k)
    v_view = (
        v.reshape(N_ROWS // 8, 8, N_COLS // 128, 128)
        .transpose(0, 2, 1, 3)
        .reshape(-1)
    )
    colq = (
        col2d.reshape(N_ROWS // STRIP // 8, 8, 1, STRIP)
        .transpose(0, 2, 1, 3)
        .reshape(-1)
    )
    return _sc_gather(v_view, colq)


# speedup vs baseline: 1.5920x; 1.0014x over previous
import functools

import jax
import jax.numpy as jnp
from jax import lax
from jax.experimental import pallas as pl
from jax.experimental.pallas import tpu as pltpu
from jax.experimental.pallas import tpu_sc as plsc

N_ROWS = 65536
N_COLS = 1024
ROW_BLOCK = 4096
STRIP = 128
LANES = 128


def _argmax_body(k_ref, idx_ref):
    lanef = lax.broadcasted_iota(jnp.int32, (STRIP, LANES), 1).astype(
        jnp.float32
    )
    rows = []
    for s in range(ROW_BLOCK // STRIP):
        r0 = s * STRIP
        mr = k_ref[pl.ds(r0, STRIP), pl.ds(0, LANES)]
        for j in range(1, N_COLS // LANES):
            mr = jnp.maximum(mr, k_ref[pl.ds(r0, STRIP), pl.ds(j * LANES, LANES)])
        jwin = jnp.full((STRIP, LANES), 8.0, jnp.float32)
        for j in range(N_COLS // LANES):
            xj = k_ref[pl.ds(r0, STRIP), pl.ds(j * LANES, LANES)]
            jwin = jnp.minimum(jwin, jnp.where(xj == mr, jnp.float32(j), 8.0))
        m = jnp.max(mr, axis=1, keepdims=True)
        comb = jnp.where(mr == m, jwin * 128.0 + lanef, 2048.0)
        amax = jnp.min(comb, axis=1, keepdims=True)
        rows.append(jnp.transpose(amax))
    idx_ref[...] = jnp.concatenate(rows, axis=0).astype(jnp.int32)


def _tc_argmax(k):
    n_strips = N_ROWS // STRIP
    blk = ROW_BLOCK // STRIP
    return pl.pallas_call(
        _argmax_body,
        grid=(N_ROWS // ROW_BLOCK,),
        in_specs=[pl.BlockSpec((ROW_BLOCK, N_COLS), lambda i: (i, 0))],
        out_specs=pl.BlockSpec((blk, STRIP), lambda i: (i, 0)),
        out_shape=jax.ShapeDtypeStruct((n_strips, STRIP), jnp.int32),
        compiler_params=pltpu.CompilerParams(
            vmem_limit_bytes=100 * 1024 * 1024,
        ),
    )(k)


def _make_sc_gather():
    info = plsc.get_sparse_core_info()
    nw = info.num_cores * info.num_subcores
    b_per_w = N_ROWS // nw
    lanes = 16
    mesh = plsc.VectorSubcoreMesh(core_axis_name="c", subcore_axis_name="s")

    @functools.partial(
        pl.kernel,
        mesh=mesh,
        out_type=jax.ShapeDtypeStruct((N_ROWS,), jnp.float32),
        scratch_types=[
            pltpu.VMEM((b_per_w,), jnp.int32),
            pltpu.VMEM((b_per_w,), jnp.int32),
            pltpu.VMEM((b_per_w,), jnp.float32),
            pltpu.SemaphoreType.DMA,
        ],
    )
    def gather(vflat_hbm, colq_hbm, out_hbm, colq_v, phys_v, vals_v, sem):
        wid = lax.axis_index("s") * info.num_cores + lax.axis_index("c")
        base = wid * b_per_w
        pltpu.sync_copy(colq_hbm.at[pl.ds(base, b_per_w)], colq_v)

        def step(i, _):
            c = colq_v[pl.ds(i * lanes, lanes)]
            r0 = (((i >> 6) << 3) + ((i >> 3) & 7)) * 128 + ((i & 7) << 4)
            r = base + r0 + lax.iota(jnp.int32, lanes)
            phys_v[pl.ds(r0, lanes)] = (
                ((r >> 3) << 13)
                + ((c >> 7) << 10)
                + ((r & 7) << 7)
                + (c & 127)
            )
            return 0

        lax.fori_loop(0, b_per_w // lanes, step, 0)

        pltpu.async_copy(vflat_hbm.at[phys_v], vals_v, sem).wait()
        pltpu.sync_copy(vals_v, out_hbm.at[pl.ds(base, b_per_w)])

    return gather


_sc_gather = _make_sc_gather()


def kernel(v, k):
    col2d = _tc_argmax(k)
    v_view = (
        v.reshape(N_ROWS // 8, 8, N_COLS // 128, 128)
        .transpose(0, 2, 1, 3)
        .reshape(-1)
    )
    colq = (
        col2d.reshape(N_ROWS // STRIP // 8, 8, 1, STRIP)
        .transpose(0, 2, 1, 3)
        .reshape(-1)
    )
    return _sc_gather(v_view, colq)
